# baseline (speedup 1.0000x reference)
import functools
import jax
import jax.numpy as jnp
from jax import lax
from jax.experimental import pallas as pl
from jax.experimental.pallas import tpu as pltpu, tpu_sc as plsc

_B = 8
_C = 19
_N = 512 * 512
_K = _N // 2
_CHUNK = 16384
_NCHUNK = _N // _CHUNK
_SEG = _N // 4
_NV = _SEG // 16


def _ce_kernel(x_ref, t_ref, out_ref):
    b = pl.program_id(0)
    c = pl.program_id(1)
    x = x_ref[0]
    t = t_ref[0]
    lse = jnp.log(jnp.sum(jnp.exp(x), axis=0, keepdims=True))
    cls = jax.lax.broadcasted_iota(jnp.int32, x.shape, 0)
    xt = jnp.sum(jnp.where(cls == t, x, 0.0), axis=0, keepdims=True)
    out_ref[0] = lse - xt


def _compute_nll(input, target):
    x = input.reshape(_B, _C, _N)
    t = target.astype(jnp.int32).reshape(_B, 1, _N)
    return pl.pallas_call(
        _ce_kernel,
        grid=(_B, _NCHUNK),
        in_specs=[
            pl.BlockSpec((1, _C, _CHUNK), lambda b, c: (b, 0, c)),
            pl.BlockSpec((1, 1, _CHUNK), lambda b, c: (b, 0, c)),
        ],
        out_specs=pl.BlockSpec(
            (1, 1, _CHUNK), lambda b, c: (b * _NCHUNK + c, 0, 0)),
        out_shape=jax.ShapeDtypeStruct((_B * _NCHUNK, 1, _CHUNK), jnp.float32),
    )(x, t).reshape(_B, _N)


def _keys_from_vals(v):
    sign = jnp.uint32(0x80000000)
    bu = lax.bitcast_convert_type(v, jnp.uint32)
    return jnp.where(v < 0.0, ~bu, bu | sign)


def _sel_kernel(vals_hbm, out_hbm, vseg, hist, merged, tmp, zvec, outv,
                shared1, shared2):
    c = lax.axis_index("c")
    s = lax.axis_index("s")
    lr = s // 4
    q = s % 4
    r = c * 4 + lr
    w = c * 16 + s
    lanes = lax.iota(jnp.int32, 16)
    kk = jnp.int32(_K)

    pltpu.sync_copy(vals_hbm.at[r, pl.ds(q * _SEG, _SEG)], vseg)

    def zero_hist(_i, _):
        hist[pl.ds(_i * 16, 16)] = jnp.zeros((16,), jnp.int32)
        return 0

    def lane_merge(_i, _):
        acc = jnp.zeros((16,), jnp.int32)
        for l in range(16):
            acc = acc + hist[pl.ds(l * 256 + _i * 16, 16)]
        merged[pl.ds(_i * 16, 16)] = acc
        return 0

    def hist_pass(shift, sel_shift, sel_val):
        lax.fori_loop(0, 256, zero_hist, 0)

        def body(_i, _):
            v = vseg[pl.ds(_i * 16, 16)]
            key = _keys_from_vals(v)
            bins = ((key >> shift) & jnp.uint32(0xFF)).astype(jnp.int32)
            idx = lanes * 256 + bins
            ones = jnp.ones((16,), jnp.int32)
            if sel_shift is None:
                plsc.addupdate_scatter(hist, [idx], ones)
            else:
                m = (key >> sel_shift).astype(jnp.int32) == sel_val
                plsc.addupdate_scatter(hist, [idx], ones, mask=m)
            return 0

        lax.fori_loop(0, _NV, body, 0)
        lax.fori_loop(0, 16, lane_merge, 0)

    def cross_merge(shared):
        pltpu.sync_copy(merged, shared.at[s])
        plsc.subcore_barrier()
        acc16 = [jnp.zeros((16,), jnp.int32)] * 16
        for j in range(4):
            pltpu.sync_copy(shared.at[lr * 4 + j], tmp)
            for g in range(16):
                acc16[g] = acc16[g] + tmp[pl.ds(g * 16, 16)]
        for g in range(16):
            merged[pl.ds(g * 16, 16)] = acc16[g]

    def suffix_find(k_need):
        gs = jnp.zeros((16,), jnp.int32)
        for g in range(16):
            tot = jnp.sum(merged[pl.ds(g * 16, 16)], axis=0)
            gs = jnp.where(lanes == g, tot, gs)
        rgs = lax.rev(gs, (0,))
        sufr = plsc.cumsum(rgs)
        suf = lax.rev(sufr, (0,))
        fr = jnp.max(plsc.all_reduce_ffs(sufr >= k_need), axis=0)
        gstar = 15 - fr
        above_g = jnp.sum(jnp.where(lanes == gstar + 1, suf, 0), axis=0)
        hbins = merged[pl.ds(gstar * 16, 16)]
        crev = plsc.cumsum(lax.rev(hbins, (0,))) + above_g
        sufh = lax.rev(crev, (0,))
        fb = jnp.max(plsc.all_reduce_ffs(crev >= k_need), axis=0)
        bstar = 15 - fb
        above_b = jnp.where(
            bstar >= 15, above_g,
            jnp.sum(jnp.where(lanes == bstar + 1, sufh, 0), axis=0))
        return gstar * 16 + bstar, above_b

    hist_pass(jnp.uint32(24), None, None)
    cross_merge(shared1)
    b1, above1 = suffix_find(kk)

    hist_pass(jnp.uint32(16), jnp.uint32(24), b1)
    cross_merge(shared2)
    b2, above2 = suffix_find(kk - above1)

    t16 = jnp.broadcast_to((b1 * 256 + b2).astype(jnp.uint32), (16,))
    sign = jnp.uint32(0x80000000)
    mask31 = jnp.uint32(0x7FFFFFFF)
    hi_key = (t16 + 1) << 16
    mid_key = (t16 << 16) | jnp.uint32(0x8000)

    def inv(u):
        ub = jnp.where(u >= sign, u & mask31, ~u)
        return lax.bitcast_convert_type(ub, jnp.float32)

    v_hi = inv(hi_key)
    v_mid = inv(mid_key)

    def body3(_i, carry):
        cnt, sm = carry
        v = vseg[pl.ds(_i * 16, 16)]
        m = v >= v_hi
        return (cnt + jnp.where(m, 1, 0).astype(jnp.int32),
                sm + jnp.where(m, v, 0.0))

    cnt, sm = lax.fori_loop(
        0, _NV, body3,
        (jnp.zeros((16,), jnp.int32), jnp.zeros((16,), jnp.float32)))
    cnt_t = jnp.sum(cnt, axis=0).astype(jnp.float32)
    sum_t = jnp.sum(sm, axis=0)

    o = jnp.where(lanes == 0, cnt_t, 0.0)
    o = jnp.where(lanes == 1, sum_t, o)
    o = jnp.where(lanes == 2, v_mid, o)
    outv[...] = o
    pltpu.sync_copy(outv, out_hbm.at[w])


def _select_sc(vals):
    mesh = plsc.VectorSubcoreMesh(core_axis_name="c", subcore_axis_name="s")
    f = pl.kernel(
        _sel_kernel,
        mesh=mesh,
        compiler_params=pltpu.CompilerParams(needs_layout_passes=False),
        out_type=jax.ShapeDtypeStruct((32, 16), jnp.float32),
        scratch_types=[
            pltpu.VMEM((_SEG,), jnp.float32),
            pltpu.VMEM((16 * 256,), jnp.int32),
            pltpu.VMEM((256,), jnp.int32),
            pltpu.VMEM((256,), jnp.int32),
            pltpu.VMEM((256,), jnp.int32),
            pltpu.VMEM((16,), jnp.float32),
            pltpu.VMEM_SHARED((16, 256), jnp.int32),
            pltpu.VMEM_SHARED((16, 256), jnp.int32),
        ],
    )
    return f(vals)


def kernel(input, target):
    vals = _compute_nll(input, target)
    o = _select_sc(vals)
    o4 = o.reshape(2, 4, 4, 16)
    cnt = o4[:, :, :, 0].sum(axis=2)
    sm = o4[:, :, :, 1].sum(axis=2)
    vmid = o4[:, :, 0, 2]
    row = sm + (_K - cnt) * vmid
    return (row.sum() / jnp.float32(_B * _K)).reshape(())
